# baseline (speedup 1.0000x reference)
import functools

import jax
import jax.numpy as jnp
from jax import lax
from jax.experimental import pallas as pl
from jax.experimental.pallas import tpu as pltpu
from jax.experimental.pallas import tpu_sc as plsc

L = 16
NFEAT = 100
NPAD = 112
K = 16
NH0 = 10
D0 = NFEAT * K


def _deepfm_body(idx_hbm, w1d_hbm, v_hbm, w0m_hbm, smalls_hbm, out_hbm,
                 idx_v, V_v, wg_v, W0_v, smalls_v, res_v,
                 sem_v, sem_w, sem_w0, sem_s):
    cid = lax.axis_index("c")
    sid = lax.axis_index("s")

    @pl.when(jnp.logical_and(cid == 0, sid == 0))
    def _():
        cp_w0 = pltpu.async_copy(w0m_hbm, W0_v, sem_w0)
        cp_s = pltpu.async_copy(smalls_hbm, smalls_v, sem_s)
        pltpu.sync_copy(idx_hbm, idx_v)
        cp_v = pltpu.async_copy(v_hbm.at[idx_v], V_v, sem_v)
        cp_w = pltpu.async_copy(w1d_hbm.at[idx_v], wg_v, sem_w)
        cp_w0.wait()
        cp_s.wait()
        cp_v.wait()
        cp_w.wait()

        zero = jnp.zeros((L,), jnp.float32)

        def loop_body(c, carry):
            s = carry[0]
            q = carry[1]
            v = V_v[c, :]
            base = c * K
            new_acc = tuple(carry[2 + j] + v * W0_v[j, pl.ds(base, K)]
                            for j in range(NH0))
            return (s + v, q + v * v) + new_acc

        init = (zero, zero) + (zero,) * NH0
        fin = lax.fori_loop(0, NFEAT, loop_body, init)
        s, q = fin[0], fin[1]
        acc = fin[2:]

        lanes = lax.iota(jnp.int32, L)
        linv = zero
        for t in range(NPAD // L):
            wc = wg_v[pl.ds(t * L, L)]
            if (t + 1) * L > NFEAT:
                wc = jnp.where(lanes < NFEAT - t * L, wc, 0.0)
            linv = linv + wc
        lin = jnp.sum(linv)

        row_b0 = smalls_v[0, :]
        h0 = zero
        for j in range(NH0):
            d = jnp.maximum(jnp.sum(acc[j]) + row_b0[j], 0.0)
            h0 = h0 + jnp.where(lanes == j, d, 0.0)
        row_b1 = smalls_v[6, :]
        h1 = zero
        for j in range(5):
            d = jnp.maximum(jnp.sum(h0 * smalls_v[1 + j, :]) + row_b1[j], 0.0)
            h1 = h1 + jnp.where(lanes == j, d, 0.0)
        row_b2 = smalls_v[10, :]
        h2 = zero
        for j in range(3):
            d = jnp.maximum(jnp.sum(h1 * smalls_v[7 + j, :]) + row_b2[j], 0.0)
            h2 = h2 + jnp.where(lanes == j, d, 0.0)
        row_w3 = smalls_v[11, :]
        dnn = jnp.sum(h2 * row_w3)
        b3s = row_w3[3]
        w0s = row_w3[4]

        pair = 0.5 * (jnp.sum(s * s) - jnp.sum(q))
        res = pair + lin + w0s + dnn + b3s
        res_v[:] = jnp.where(lanes == 0, res, 0.0)
        pltpu.sync_copy(res_v.at[pl.ds(0, 1)], out_hbm)


_deepfm_sc = functools.partial(
    pl.kernel,
    out_type=jax.ShapeDtypeStruct((1,), jnp.float32),
    mesh=plsc.VectorSubcoreMesh(core_axis_name="c", subcore_axis_name="s"),
    compiler_params=pltpu.CompilerParams(needs_layout_passes=False,
                                         use_tc_tiling_on_sc=False),
    scratch_types=[
        pltpu.VMEM((NPAD,), jnp.int32),
        pltpu.VMEM((NPAD, K), jnp.float32),
        pltpu.VMEM((NPAD,), jnp.float32),
        pltpu.VMEM((NH0, D0), jnp.float32),
        pltpu.VMEM((12, L), jnp.float32),
        pltpu.VMEM((L,), jnp.float32),
        pltpu.SemaphoreType.DMA,
        pltpu.SemaphoreType.DMA,
        pltpu.SemaphoreType.DMA,
        pltpu.SemaphoreType.DMA,
    ],
)(_deepfm_body)


def kernel(feature, w_table, v_table, w0, W0, b0, W1, b1, W2, b2, W3, b3):
    feature = feature.astype(jnp.int32)
    idx = jnp.concatenate([feature, jnp.zeros((NPAD - NFEAT,), jnp.int32)])
    w1d = w_table.reshape(-1)
    row_b0 = jnp.pad(b0, (0, L - NH0))
    w1_rows = jnp.pad(W1, ((0, 0), (0, L - NH0)))
    row_b1 = jnp.pad(b1, (0, L - 5))
    w2_rows = jnp.pad(W2, ((0, 0), (0, L - 5)))
    row_b2 = jnp.pad(b2, (0, L - 3))
    row_w3 = jnp.concatenate([W3[0], b3, w0, jnp.zeros((L - 5,), jnp.float32)])
    smalls = jnp.concatenate(
        [row_b0[None], w1_rows, row_b1[None], w2_rows, row_b2[None],
         row_w3[None]], axis=0)
    return _deepfm_sc(idx, w1d, v_table, W0, smalls)
